# baseline (speedup 1.0000x reference)
import jax
import jax.numpy as jnp
from jax.experimental import pallas as pl


def _gcn_conv(x, edge_index, W, b):
    n = x.shape[0]
    loop = jnp.arange(n, dtype=edge_index.dtype)
    src = jnp.concatenate([edge_index[0], loop])
    dst = jnp.concatenate([edge_index[1], loop])
    deg = jnp.zeros((n,), x.dtype).at[dst].add(1.0)
    dinv = jnp.where(deg > 0, 1.0 / jnp.sqrt(deg), 0.0)
    norm = dinv[src] * dinv[dst]
    h = x @ W
    out = jnp.zeros((n, W.shape[1]), x.dtype).at[dst].add(h[src] * norm[:, None])
    return out + b


def _relu_kernel(x_ref, o_ref):
    o_ref[...] = jnp.maximum(x_ref[...], 0.0)


def _prelu(x):
    return pl.pallas_call(
        _relu_kernel,
        out_shape=jax.ShapeDtypeStruct(x.shape, x.dtype),
    )(x)


def kernel(adj, features, W1, b1, W2, b2, W3, b3, W4, b4, W5, b5, W6, b6, W7, b7, in_proj_w, in_proj_b, out_proj_w, out_proj_b):
    h = _prelu(_gcn_conv(features, adj, W1, b1))
    h = _prelu(_gcn_conv(h, adj, W2, b2))
    h = _prelu(_gcn_conv(h, adj, W3, b3))
    Ed = h.shape[1]
    v = h @ in_proj_w[2 * Ed:].T + in_proj_b[2 * Ed:]
    h = v @ out_proj_w.T + out_proj_b
    h = _prelu(_gcn_conv(h, adj, W4, b4))
    h = _prelu(_gcn_conv(h, adj, W5, b5))
    h = _prelu(_gcn_conv(h, adj, W6, b6))
    h = _prelu(_gcn_conv(h, adj, W7, b7))
    return h


# speedup vs baseline: 31.2211x; 31.2211x over previous
import functools

import jax
import jax.numpy as jnp
from jax import lax
from jax.experimental import pallas as pl
from jax.experimental.pallas import tpu as pltpu
from jax.experimental.pallas import tpu_sc as plsc

_N = 10000
_NP = 10240
_E = 320000
_CH = 128
_NW = 32
_CPW = 80
_NCHUNK = _NW * _CPW
_EP = _NCHUNK * _CH
_RPS = _NP // 16
_LAST = _N - 15 * _RPS
_BN = 2000

_mesh = plsc.VectorSubcoreMesh(core_axis_name="c", subcore_axis_name="s")
_sc_params = pltpu.CompilerParams(use_tc_tiling_on_sc=False)


def _copy_out(acc, out_hbm, c, s):
    @pl.when(s < 15)
    def _():
        pltpu.sync_copy(acc.at[pl.ds(s * _RPS, _RPS)],
                        out_hbm.at[c, pl.ds(s * _RPS, _RPS)])

    @pl.when(s == 15)
    def _():
        pltpu.sync_copy(acc.at[pl.ds(15 * _RPS, _LAST)],
                        out_hbm.at[c, pl.ds(15 * _RPS, _LAST)])


def _zero_acc(zbuf, acc, s, f):
    @pl.loop(0, 32)
    def _(i):
        for j in range(f // 16):
            zbuf[i, pl.ds(j * 16, 16)] = jnp.zeros((16,), jnp.float32)

    for r in range(_RPS // 32):
        pltpu.sync_copy(zbuf, acc.at[pl.ds(s * _RPS + r * 32, 32)])


@functools.partial(
    pl.kernel,
    out_type=jax.ShapeDtypeStruct((2, _N, 16), jnp.float32),
    mesh=_mesh,
    compiler_params=_sc_params,
    scratch_types=[
        pltpu.VMEM((_CPW, _CH), jnp.int32),
        pltpu.VMEM((_CH, 16), jnp.float32),
        pltpu.VMEM((32, 16), jnp.float32),
        pltpu.VMEM_SHARED((_NP, 16), jnp.float32),
        pltpu.SemaphoreType.DMA,
    ],
)
def _sc_degree(dst_hbm, out_hbm, dstb, ones, zbuf, acc, ssem):
    c = lax.axis_index("c")
    s = lax.axis_index("s")
    w = s * 2 + c

    @pl.loop(0, _CH)
    def _(i):
        ones[i, pl.ds(0, 16)] = jnp.ones((16,), jnp.float32)

    _zero_acc(zbuf, acc, s, 16)
    pltpu.sync_copy(dst_hbm.at[pl.ds(w * _CPW, _CPW)], dstb)
    plsc.subcore_barrier()

    G = 8
    prev = None
    for t in range(_CPW // G):
        if prev is not None:
            for d in prev:
                d.wait()
        prev = [
            pltpu.async_copy(ones, acc.at[dstb.at[t * G + g]], ssem, add=True)
            for g in range(G)
        ]
    for d in prev:
        d.wait()

    plsc.subcore_barrier()
    _copy_out(acc, out_hbm, c, s)


def _stage_table(h_hbm, tab, s):
    @pl.when(s < 15)
    def _():
        pltpu.sync_copy(h_hbm.at[pl.ds(s * _RPS, _RPS)],
                        tab.at[pl.ds(s * _RPS, _RPS)])

    @pl.when(s == 15)
    def _():
        pltpu.sync_copy(h_hbm.at[pl.ds(15 * _RPS, _LAST)],
                        tab.at[pl.ds(15 * _RPS, _LAST)])


def _make_agg(f):
    G = 32768 // (_CH * f)
    NT = _CPW // G

    @functools.partial(
        pl.kernel,
        out_type=jax.ShapeDtypeStruct((2, _N, f), jnp.float32),
        mesh=_mesh,
        compiler_params=_sc_params,
        scratch_types=[
            pltpu.VMEM((_CPW, _CH), jnp.int32),
            pltpu.VMEM((_CPW, _CH), jnp.int32),
            pltpu.VMEM((G * _CH, f), jnp.float32),
            pltpu.VMEM((G * _CH, f), jnp.float32),
            pltpu.VMEM((32, f), jnp.float32),
            pltpu.VMEM_SHARED((_NP, f), jnp.float32),
            pltpu.VMEM_SHARED((_NP, f), jnp.float32),
            pltpu.SemaphoreType.DMA,
            pltpu.SemaphoreType.DMA,
            pltpu.SemaphoreType.DMA,
            pltpu.SemaphoreType.DMA,
        ],
    )
    def agg(src_hbm, dst_hbm, h_hbm, out_hbm, srcb, dstb, rows0, rows1,
            zbuf, acc, tab, gsem0, gsem1, ssem0, ssem1):
        c = lax.axis_index("c")
        s = lax.axis_index("s")
        w = s * 2 + c

        _zero_acc(zbuf, acc, s, f)
        _stage_table(h_hbm, tab, s)
        pltpu.sync_copy(src_hbm.at[pl.ds(w * _CPW, _CPW)], srcb)
        pltpu.sync_copy(dst_hbm.at[pl.ds(w * _CPW, _CPW)], dstb)
        plsc.subcore_barrier()

        rows = (rows0, rows1)
        gsem = (gsem0, gsem1)
        ssem = (ssem0, ssem1)

        def fire_gathers(t):
            b = t % 2
            return [
                pltpu.async_copy(tab.at[srcb.at[t * G + g]],
                                 rows[b].at[pl.ds(g * _CH, _CH)], gsem[b])
                for g in range(G)
            ]

        def fire_scatters(t):
            b = t % 2
            return [
                pltpu.async_copy(rows[b].at[pl.ds(g * _CH, _CH)],
                                 acc.at[dstb.at[t * G + g]], ssem[b],
                                 add=True)
                for g in range(G)
            ]

        def drain(ds):
            for d in ds:
                d.wait()

        g_in = fire_gathers(0)
        s_in = {}
        for t in range(NT):
            if t >= 1:
                drain(s_in.pop(t - 1))
            if t + 1 < NT:
                g_next = fire_gathers(t + 1)
            drain(g_in)
            s_in[t] = fire_scatters(t)
            if t + 1 < NT:
                g_in = g_next
        drain(s_in.pop(NT - 1))

        plsc.subcore_barrier()
        _copy_out(acc, out_hbm, c, s)

    return agg


_agg32 = _make_agg(32)


def _aggregate(src, dst, h):
    f = h.shape[1]
    return [_agg32(src, dst, h[:, k * 32:(k + 1) * 32])
            for k in range(f // 32)]


def _row_spec(f):
    return pl.BlockSpec((_BN, f), lambda i: (i, 0))


def _part_spec(f):
    return pl.BlockSpec((2, _BN, f), lambda i: (0, i, 0))


def _full_spec(shape):
    nd = len(shape)
    return pl.BlockSpec(shape, lambda i, _n=nd: (0,) * _n)


def _tc_matmul(x, W):
    fi, fo = W.shape

    def body(x_ref, w_ref, o_ref):
        o_ref[...] = jnp.dot(x_ref[...], w_ref[...],
                             preferred_element_type=jnp.float32)

    return pl.pallas_call(
        body,
        grid=(_N // _BN,),
        in_specs=[_row_spec(fi), _full_spec(W.shape)],
        out_specs=_row_spec(fo),
        out_shape=jax.ShapeDtypeStruct((_N, fo), jnp.float32),
    )(x, W)


def _tc_dinv_scale(degp, h):
    f = h.shape[1]

    def body(degp_ref, h_ref, dinv_ref, hs_ref):
        deg = degp_ref[0, :, 0:1] + degp_ref[1, :, 0:1] + 1.0
        d = lax.rsqrt(deg)
        dinv_ref[...] = d
        hs_ref[...] = d * h_ref[...]

    return pl.pallas_call(
        body,
        grid=(_N // _BN,),
        in_specs=[_part_spec(16), _row_spec(f)],
        out_specs=[_row_spec(1), _row_spec(f)],
        out_shape=[jax.ShapeDtypeStruct((_N, 1), jnp.float32),
                   jax.ShapeDtypeStruct((_N, f), jnp.float32)],
    )(degp, h)


def _sum_slabs(p_refs):
    parts = [pr[0] + pr[1] for pr in p_refs]
    return parts[0] if len(parts) == 1 else jnp.concatenate(parts, axis=1)


def _tc_post(ps, hs, dinv, b, W):
    f = hs.shape[1]
    fo = W.shape[1]
    k = len(ps)

    def body(*refs):
        p_refs, (hs_ref, d_ref, b_ref, w_ref, o_ref) = refs[:k], refs[k:]
        d = d_ref[...]
        y = jnp.maximum(d * (_sum_slabs(p_refs) + hs_ref[...]) + b_ref[...],
                        0.0)
        o_ref[...] = d * jnp.dot(y, w_ref[...],
                                 preferred_element_type=jnp.float32)

    return pl.pallas_call(
        body,
        grid=(_N // _BN,),
        in_specs=[_part_spec(32)] * k + [_row_spec(f), _row_spec(1),
                  _full_spec(b.shape), _full_spec(W.shape)],
        out_specs=_row_spec(fo),
        out_shape=jax.ShapeDtypeStruct((_N, fo), jnp.float32),
    )(*ps, hs, dinv, b, W)


def _tc_post_mha(p, hs, dinv, b, wv, bv, wo, bo):
    f = hs.shape[1]

    def body(p_ref, hs_ref, d_ref, b_ref, wv_ref, bv_ref, wo_ref, bo_ref,
             o_ref):
        d = d_ref[...]
        y = jnp.maximum(d * (_sum_slabs([p_ref]) + hs_ref[...]) + b_ref[...],
                        0.0)
        v = lax.dot_general(y, wv_ref[...], (((1,), (1,)), ((), ())),
                            preferred_element_type=jnp.float32) + bv_ref[...]
        z = lax.dot_general(v, wo_ref[...], (((1,), (1,)), ((), ())),
                            preferred_element_type=jnp.float32) + bo_ref[...]
        o_ref[...] = d * z

    return pl.pallas_call(
        body,
        grid=(_N // _BN,),
        in_specs=[_part_spec(f), _row_spec(f), _row_spec(1),
                  _full_spec(b.shape), _full_spec(wv.shape),
                  _full_spec(bv.shape), _full_spec(wo.shape),
                  _full_spec(bo.shape)],
        out_specs=_row_spec(f),
        out_shape=jax.ShapeDtypeStruct((_N, f), jnp.float32),
    )(p, hs, dinv, b, wv, bv, wo, bo)


def _tc_pre(ps, u, dinv, W, b, final):
    f = u.shape[1]
    fo = W.shape[1]
    k = len(ps)

    def body(*refs):
        p_refs, (u_ref, d_ref, w_ref, b_ref, o_ref) = refs[:k], refs[k:]
        d = d_ref[...]
        t = d * (_sum_slabs(p_refs) + u_ref[...])
        y = jnp.maximum(jnp.dot(t, w_ref[...],
                                preferred_element_type=jnp.float32)
                        + b_ref[...], 0.0)
        o_ref[...] = y if final else d * y

    return pl.pallas_call(
        body,
        grid=(_N // _BN,),
        in_specs=[_part_spec(32)] * k + [_row_spec(f), _row_spec(1),
                  _full_spec(W.shape), _full_spec(b.shape)],
        out_specs=_row_spec(fo),
        out_shape=jax.ShapeDtypeStruct((_N, fo), jnp.float32),
    )(*ps, u, dinv, W, b)


def kernel(adj, features, W1, b1, W2, b2, W3, b3, W4, b4, W5, b5, W6, b6,
           W7, b7, in_proj_w, in_proj_b, out_proj_w, out_proj_b):
    b1r, b2r, b3r = b1[None, :], b2[None, :], b3[None, :]
    b6r, b7r = b6[None, :], b7[None, :]
    Ed = 32
    wv = in_proj_w[2 * Ed:]
    bv = in_proj_b[None, 2 * Ed:]
    bo = out_proj_b[None, :]
    W4p = jnp.pad(W4, ((0, 0), (0, 8)))
    b4p = jnp.pad(b4, (0, 8))[None, :]
    W5p = jnp.pad(W5, ((0, 8), (0, 0)))
    b5r = b5[None, :]

    npad = _EP - _E
    src = jnp.concatenate(
        [adj[0], jnp.zeros((npad,), adj.dtype)]).reshape(_NCHUNK, _CH)
    dst = jnp.concatenate(
        [adj[1],
         _N + (jnp.arange(npad, dtype=adj.dtype) % (_NP - _N))]
    ).reshape(_NCHUNK, _CH)

    degp = _sc_degree(dst)
    h1 = _tc_matmul(features, W1)
    dinv, hs1 = _tc_dinv_scale(degp, h1)

    p1 = _aggregate(src, dst, hs1)
    hs2 = _tc_post(p1, hs1, dinv, b1r, W2)
    p2 = _aggregate(src, dst, hs2)
    hs3 = _tc_post(p2, hs2, dinv, b2r, W3)
    p3 = _aggregate(src, dst, hs3)
    u4 = _tc_post_mha(p3[0], hs3, dinv, b3r, wv, bv, out_proj_w, bo)
    p4 = _aggregate(src, dst, u4)
    u5 = _tc_pre(p4, u4, dinv, W4p, b4p, final=False)
    p5 = _aggregate(src, dst, u5)
    u6 = _tc_pre(p5, u5, dinv, W5p, b5r, final=False)
    p6 = _aggregate(src, dst, u6)
    u7 = _tc_pre(p6, u6, dinv, W6, b6r, final=False)
    p7 = _aggregate(src, dst, u7)
    out = _tc_pre(p7, u7, dinv, W7, b7r, final=True)
    return out
